# baseline (speedup 1.0000x reference)
import functools

import jax
import jax.numpy as jnp
from jax import lax
from jax.experimental import pallas as pl
from jax.experimental.pallas import tpu as pltpu
from jax.experimental.pallas import tpu_sc as plsc

_B, _S, _D = 4, 4096, 1024
_N = _B * _S
_NW = 32
_SPW = _S // _NW
_C = 16
_NSC = _SPW // _C
_STEPS = _NSC * _B
_NBUF = 5
_LANES = 16


def _embed_body(ids_hbm, tok_hbm, pos_hbm, out_hbm,
                idx_v, tbufs, pbufs, gsems, psems, ssems):
    cid = lax.axis_index("c")
    sid = lax.axis_index("s")
    wid = sid * 2 + cid
    s_base = wid * _SPW

    for b in range(_B):
        pltpu.sync_copy(ids_hbm.at[pl.ds(b * _S + s_base, _SPW)],
                        idx_v.at[pl.ds(b * _SPW, _SPW)])

    def start_gather(t):
        sc, b = t // _B, t % _B
        idx = idx_v.at[pl.ds(b * _SPW + sc * _C, _C)]
        return pltpu.async_copy(tok_hbm.at[idx], tbufs[t % _NBUF],
                                gsems[t % _NBUF])

    def start_pos(sc):
        return pltpu.async_copy(pos_hbm.at[pl.ds(s_base + sc * _C, _C)],
                                pbufs[sc % 2], psems[sc % 2])

    g_desc = [None] * _STEPS
    s_desc = [None] * _STEPS
    p_desc = [None] * _NSC
    p_desc[0] = start_pos(0)
    for t in range(_NBUF - 1):
        g_desc[t] = start_gather(t)

    for t in range(_STEPS):
        sc, b = t // _B, t % _B
        if b == 0:
            if sc + 1 < _NSC:
                p_desc[sc + 1] = start_pos(sc + 1)
            p_desc[sc].wait()
        g_desc[t].wait()

        tbuf, pbuf = tbufs[t % _NBUF], pbufs[sc % 2]

        def add_row(r, carry, tbuf=tbuf, pbuf=pbuf):
            for k in range(_D // _LANES):
                sl = pl.ds(k * _LANES, _LANES)
                tbuf[r, sl] = tbuf[r, sl] + pbuf[r, sl]
            return carry
        lax.fori_loop(0, _C, add_row, 0)

        s_desc[t] = pltpu.async_copy(
            tbuf, out_hbm.at[pl.ds(b * _S + s_base + sc * _C, _C)],
            ssems[t % _NBUF])

        if t + _NBUF - 1 < _STEPS:
            if t >= 1:
                s_desc[t - 1].wait()
            g_desc[t + _NBUF - 1] = start_gather(t + _NBUF - 1)

    for t in range(_STEPS - _NBUF + 1, _STEPS):
        s_desc[t].wait()


_embed_kernel = functools.partial(
    pl.kernel,
    out_type=jax.ShapeDtypeStruct((_N, _D), jnp.float32),
    mesh=plsc.VectorSubcoreMesh(core_axis_name="c", subcore_axis_name="s"),
    scratch_types=[
        pltpu.VMEM((_B * _SPW,), jnp.int32),
        tuple(pltpu.VMEM((_C, _D), jnp.float32) for _ in range(_NBUF)),
        tuple(pltpu.VMEM((_C, _D), jnp.float32) for _ in range(2)),
        tuple(pltpu.SemaphoreType.DMA for _ in range(_NBUF)),
        tuple(pltpu.SemaphoreType.DMA for _ in range(2)),
        tuple(pltpu.SemaphoreType.DMA for _ in range(_NBUF)),
    ],
)(_embed_body)


def kernel(input_ids, tok_table, pos_table):
    ids = input_ids.reshape(-1).astype(jnp.int32)
    out = _embed_kernel(ids, tok_table, pos_table)
    return out.reshape(_B, _S, _D)


# speedup vs baseline: 1.0011x; 1.0011x over previous
import functools

import jax
import jax.numpy as jnp
from jax import lax
from jax.experimental import pallas as pl
from jax.experimental.pallas import tpu as pltpu
from jax.experimental.pallas import tpu_sc as plsc

_B, _S, _D = 4, 4096, 1024
_N = _B * _S
_NW = 32
_SPW = _S // _NW
_C = 16
_NSC = _SPW // _C
_STEPS = _NSC * _B
_NBUF = 4
_LANES = 16


def _embed_body(ids_hbm, tok_hbm, pos_hbm, out_hbm,
                idx_v, tbufs, pbufs, gsems, psems, ssems):
    cid = lax.axis_index("c")
    sid = lax.axis_index("s")
    wid = sid * 2 + cid
    s_base = wid * _SPW

    for b in range(_B):
        pltpu.sync_copy(ids_hbm.at[pl.ds(b * _S + s_base, _SPW)],
                        idx_v.at[pl.ds(b * _SPW, _SPW)])

    def start_gather(t):
        sc, b = t // _B, t % _B
        idx = idx_v.at[pl.ds(b * _SPW + sc * _C, _C)]
        return pltpu.async_copy(tok_hbm.at[idx], tbufs[t % _NBUF],
                                gsems[t % _NBUF])

    def start_pos(sc):
        return pltpu.async_copy(pos_hbm.at[pl.ds(s_base + sc * _C, _C)],
                                pbufs[sc % 2], psems[sc % 2])

    g_desc = [None] * _STEPS
    s_desc = [None] * _STEPS
    p_desc = [None] * _NSC
    p_desc[0] = start_pos(0)
    for t in range(_NBUF - 1):
        g_desc[t] = start_gather(t)

    for t in range(_STEPS):
        sc, b = t // _B, t % _B
        if b == 0:
            if sc + 1 < _NSC:
                p_desc[sc + 1] = start_pos(sc + 1)
            p_desc[sc].wait()
        g_desc[t].wait()

        tbuf, pbuf = tbufs[t % _NBUF], pbufs[sc % 2]

        def add_row(r, carry, tbuf=tbuf, pbuf=pbuf):
            for k in range(_D // _LANES):
                sl = pl.ds(k * _LANES, _LANES)
                tbuf[r, sl] = tbuf[r, sl] + pbuf[r, sl]
            return carry
        lax.fori_loop(0, _C, add_row, 0)

        s_desc[t] = pltpu.async_copy(
            tbuf, out_hbm.at[pl.ds(b * _S + s_base + sc * _C, _C)],
            ssems[t % _NBUF])

        if t + _NBUF - 1 < _STEPS:
            if t >= 1:
                s_desc[t - 1].wait()
            g_desc[t + _NBUF - 1] = start_gather(t + _NBUF - 1)

    for t in range(_STEPS - _NBUF + 1, _STEPS):
        s_desc[t].wait()


_embed_kernel = functools.partial(
    pl.kernel,
    out_type=jax.ShapeDtypeStruct((_N, _D), jnp.float32),
    mesh=plsc.VectorSubcoreMesh(core_axis_name="c", subcore_axis_name="s"),
    scratch_types=[
        pltpu.VMEM((_B * _SPW,), jnp.int32),
        tuple(pltpu.VMEM((_C, _D), jnp.float32) for _ in range(_NBUF)),
        tuple(pltpu.VMEM((_C, _D), jnp.float32) for _ in range(2)),
        tuple(pltpu.SemaphoreType.DMA for _ in range(_NBUF)),
        tuple(pltpu.SemaphoreType.DMA for _ in range(2)),
        tuple(pltpu.SemaphoreType.DMA for _ in range(_NBUF)),
    ],
)(_embed_body)


def kernel(input_ids, tok_table, pos_table):
    ids = input_ids.reshape(-1).astype(jnp.int32)
    out = _embed_kernel(ids, tok_table, pos_table)
    return out.reshape(_B, _S, _D)


# speedup vs baseline: 1.1022x; 1.1010x over previous
import functools

import jax
import jax.numpy as jnp
from jax import lax
from jax.experimental import pallas as pl
from jax.experimental.pallas import tpu as pltpu
from jax.experimental.pallas import tpu_sc as plsc

_B, _S, _D = 4, 4096, 1024
_N = _B * _S
_NW = 32
_SPW = _S // _NW
_C = 16
_NSC = _SPW // _C
_STEPS = _NSC * _B
_NBUF = 4
_LANES = 16


def _embed_body(ids_hbm, tok_hbm, pos_hbm, out_hbm,
                idx_v, tbufs, pbufs, gsems, psems, ssems):
    cid = lax.axis_index("c")
    sid = lax.axis_index("s")
    wid = sid * 2 + cid
    s_base = wid * _SPW

    for b in range(_B):
        pltpu.sync_copy(ids_hbm.at[pl.ds(b * _S + s_base, _SPW)],
                        idx_v.at[pl.ds(b * _SPW, _SPW)])

    def start_gather(t):
        sc, b = t // _B, t % _B
        idx = idx_v.at[pl.ds(b * _SPW + sc * _C, _C)]
        return pltpu.async_copy(tok_hbm.at[idx], tbufs[t % _NBUF],
                                gsems[t % _NBUF])

    def start_pos(sc):
        return pltpu.async_copy(pos_hbm.at[pl.ds(s_base + sc * _C, _C)],
                                pbufs[sc % 2], psems[sc % 2])

    g_desc = [None] * _STEPS
    s_desc = [None] * _STEPS
    p_desc = [None] * _NSC
    p_desc[0] = start_pos(0)
    for t in range(_NBUF - 1):
        g_desc[t] = start_gather(t)

    for t in range(_STEPS):
        sc, b = t // _B, t % _B
        if b == 0:
            if sc + 1 < _NSC:
                p_desc[sc + 1] = start_pos(sc + 1)
            p_desc[sc].wait()
        g_desc[t].wait()

        tbuf, pbuf = tbufs[t % _NBUF], pbufs[sc % 2]

        def add_half(i, carry, tbuf=tbuf, pbuf=pbuf):
            r = i // 2
            h = (i % 2) * (_D // 2)
            for k in range(_D // (2 * _LANES)):
                sl = pl.ds(h + k * _LANES, _LANES)
                tbuf[r, sl] = tbuf[r, sl] + pbuf[r, sl]
            return carry
        lax.fori_loop(0, 2 * _C, add_half, 0)

        s_desc[t] = pltpu.async_copy(
            tbuf, out_hbm.at[pl.ds(b * _S + s_base + sc * _C, _C)],
            ssems[t % _NBUF])

        if t + _NBUF - 1 < _STEPS:
            if t >= 1:
                s_desc[t - 1].wait()
            g_desc[t + _NBUF - 1] = start_gather(t + _NBUF - 1)

    for t in range(_STEPS - _NBUF + 1, _STEPS):
        s_desc[t].wait()


_embed_kernel = functools.partial(
    pl.kernel,
    out_type=jax.ShapeDtypeStruct((_N, _D), jnp.float32),
    mesh=plsc.VectorSubcoreMesh(core_axis_name="c", subcore_axis_name="s"),
    scratch_types=[
        pltpu.VMEM((_B * _SPW,), jnp.int32),
        tuple(pltpu.VMEM((_C, _D), jnp.float32) for _ in range(_NBUF)),
        tuple(pltpu.VMEM((_C, _D), jnp.float32) for _ in range(2)),
        tuple(pltpu.SemaphoreType.DMA for _ in range(_NBUF)),
        tuple(pltpu.SemaphoreType.DMA for _ in range(2)),
        tuple(pltpu.SemaphoreType.DMA for _ in range(_NBUF)),
    ],
)(_embed_body)


def kernel(input_ids, tok_table, pos_table):
    ids = input_ids.reshape(-1).astype(jnp.int32)
    out = _embed_kernel(ids, tok_table, pos_table)
    return out.reshape(_B, _S, _D)


# speedup vs baseline: 1.1476x; 1.0412x over previous
import functools

import jax
import jax.numpy as jnp
from jax import lax
from jax.experimental import pallas as pl
from jax.experimental.pallas import tpu as pltpu
from jax.experimental.pallas import tpu_sc as plsc

_B, _S, _D = 4, 4096, 1024
_N = _B * _S
_NW = 32
_SPW = _S // _NW
_C = 16
_NSC = _SPW // _C
_STEPS = _NSC * _B
_NBUF = 4
_LANES = 16


def _embed_body(ids_hbm, tok_hbm, pos_hbm, out_hbm,
                idx_v, tbufs, pbufs, gsems, psems, ssems):
    cid = lax.axis_index("c")
    sid = lax.axis_index("s")
    wid = sid * 2 + cid
    s_base = wid * _SPW

    for b in range(_B):
        pltpu.sync_copy(ids_hbm.at[pl.ds(b * _S + s_base, _SPW)],
                        idx_v.at[pl.ds(b * _SPW, _SPW)])

    def start_gather(t):
        sc, b = t // _B, t % _B
        idx = idx_v.at[pl.ds(b * _SPW + sc * _C, _C)]
        return pltpu.async_copy(tok_hbm.at[idx], tbufs[t % _NBUF],
                                gsems[t % _NBUF])

    def start_pos(sc):
        return pltpu.async_copy(pos_hbm.at[pl.ds(s_base + sc * _C, _C)],
                                pbufs[sc % 2], psems[sc % 2])

    g_desc = [None] * _STEPS
    s_desc = [None] * _STEPS
    p_desc = [None] * _NSC
    p_desc[0] = start_pos(0)
    for t in range(_NBUF - 1):
        g_desc[t] = start_gather(t)

    for t in range(_STEPS):
        sc, b = t // _B, t % _B
        if b == 0:
            if sc + 1 < _NSC:
                p_desc[sc + 1] = start_pos(sc + 1)
            p_desc[sc].wait()
        g_desc[t].wait()

        tbuf, pbuf = tbufs[t % _NBUF], pbufs[sc % 2]

        def add_quarter(i, carry, tbuf=tbuf, pbuf=pbuf):
            r = i // 4
            h = (i % 4) * (_D // 4)
            for k in range(_D // (4 * _LANES)):
                sl = pl.ds(h + k * _LANES, _LANES)
                tbuf[r, sl] = tbuf[r, sl] + pbuf[r, sl]
            return carry
        lax.fori_loop(0, 4 * _C, add_quarter, 0)

        s_desc[t] = pltpu.async_copy(
            tbuf, out_hbm.at[pl.ds(b * _S + s_base + sc * _C, _C)],
            ssems[t % _NBUF])

        if t + _NBUF - 1 < _STEPS:
            if t >= 1:
                s_desc[t - 1].wait()
            g_desc[t + _NBUF - 1] = start_gather(t + _NBUF - 1)

    for t in range(_STEPS - _NBUF + 1, _STEPS):
        s_desc[t].wait()


_embed_kernel = functools.partial(
    pl.kernel,
    out_type=jax.ShapeDtypeStruct((_N, _D), jnp.float32),
    mesh=plsc.VectorSubcoreMesh(core_axis_name="c", subcore_axis_name="s"),
    scratch_types=[
        pltpu.VMEM((_B * _SPW,), jnp.int32),
        tuple(pltpu.VMEM((_C, _D), jnp.float32) for _ in range(_NBUF)),
        tuple(pltpu.VMEM((_C, _D), jnp.float32) for _ in range(2)),
        tuple(pltpu.SemaphoreType.DMA for _ in range(_NBUF)),
        tuple(pltpu.SemaphoreType.DMA for _ in range(2)),
        tuple(pltpu.SemaphoreType.DMA for _ in range(_NBUF)),
    ],
)(_embed_body)


def kernel(input_ids, tok_table, pos_table):
    ids = input_ids.reshape(-1).astype(jnp.int32)
    out = _embed_kernel(ids, tok_table, pos_table)
    return out.reshape(_B, _S, _D)


# speedup vs baseline: 1.1805x; 1.0286x over previous
import functools

import jax
import jax.numpy as jnp
from jax import lax
from jax.experimental import pallas as pl
from jax.experimental.pallas import tpu as pltpu
from jax.experimental.pallas import tpu_sc as plsc

_B, _S, _D = 4, 4096, 1024
_N = _B * _S
_NW = 32
_SPW = _S // _NW
_C = 16
_NSC = _SPW // _C
_STEPS = _NSC * _B
_NBUF = 4
_LANES = 16


def _embed_body(ids_hbm, tok_hbm, pos_hbm, out_hbm,
                idx_v, tbufs, pbufs, gsems, psems, ssems):
    cid = lax.axis_index("c")
    sid = lax.axis_index("s")
    wid = sid * 2 + cid
    s_base = wid * _SPW

    for b in range(_B):
        pltpu.sync_copy(ids_hbm.at[pl.ds(b * _S + s_base, _SPW)],
                        idx_v.at[pl.ds(b * _SPW, _SPW)])

    def start_gather(t):
        sc, b = t // _B, t % _B
        idx = idx_v.at[pl.ds(b * _SPW + sc * _C, _C)]
        return pltpu.async_copy(tok_hbm.at[idx], tbufs[t % _NBUF],
                                gsems[t % _NBUF])

    def start_pos(sc):
        return pltpu.async_copy(pos_hbm.at[pl.ds(s_base + sc * _C, _C)],
                                pbufs[sc % 2], psems[sc % 2])

    g_desc = [None] * _STEPS
    s_desc = [None] * _STEPS
    p_desc = [None] * _NSC
    p_desc[0] = start_pos(0)
    for t in range(_NBUF - 1):
        g_desc[t] = start_gather(t)

    for t in range(_STEPS):
        sc, b = t // _B, t % _B
        if b == 0:
            if sc + 1 < _NSC:
                p_desc[sc + 1] = start_pos(sc + 1)
            p_desc[sc].wait()
        g_desc[t].wait()

        tbuf, pbuf = tbufs[t % _NBUF], pbufs[sc % 2]

        def add_part(i, carry, tbuf=tbuf, pbuf=pbuf):
            r = i // 8
            h = (i % 8) * (_D // 8)
            for k in range(_D // (8 * _LANES)):
                sl = pl.ds(h + k * _LANES, _LANES)
                tbuf[r, sl] = tbuf[r, sl] + pbuf[r, sl]
            return carry
        lax.fori_loop(0, 8 * _C, add_part, 0)

        s_desc[t] = pltpu.async_copy(
            tbuf, out_hbm.at[pl.ds(b * _S + s_base + sc * _C, _C)],
            ssems[t % _NBUF])

        if t + _NBUF - 1 < _STEPS:
            if t >= 1:
                s_desc[t - 1].wait()
            g_desc[t + _NBUF - 1] = start_gather(t + _NBUF - 1)

    for t in range(_STEPS - _NBUF + 1, _STEPS):
        s_desc[t].wait()


_embed_kernel = functools.partial(
    pl.kernel,
    out_type=jax.ShapeDtypeStruct((_N, _D), jnp.float32),
    mesh=plsc.VectorSubcoreMesh(core_axis_name="c", subcore_axis_name="s"),
    scratch_types=[
        pltpu.VMEM((_B * _SPW,), jnp.int32),
        tuple(pltpu.VMEM((_C, _D), jnp.float32) for _ in range(_NBUF)),
        tuple(pltpu.VMEM((_C, _D), jnp.float32) for _ in range(2)),
        tuple(pltpu.SemaphoreType.DMA for _ in range(_NBUF)),
        tuple(pltpu.SemaphoreType.DMA for _ in range(2)),
        tuple(pltpu.SemaphoreType.DMA for _ in range(_NBUF)),
    ],
)(_embed_body)


def kernel(input_ids, tok_table, pos_table):
    ids = input_ids.reshape(-1).astype(jnp.int32)
    out = _embed_kernel(ids, tok_table, pos_table)
    return out.reshape(_B, _S, _D)
